# baseline (speedup 1.0000x reference)
import jax
import jax.numpy as jnp
from jax.experimental import pallas as pl
from jax.experimental.pallas import tpu as pltpu

_BM = 1024
_BI = 1024


_NSPLIT = 4


def _ffn_body(x_ref, wg_ref, wu_ref, wd_ref, out_ref, xb_s):
    i = pl.program_id(1)

    @pl.when(i == 0)
    def _cast_x():
        xb_s[...] = x_ref[...].astype(jnp.bfloat16)

    xb = xb_s[...]
    nt = (((1,), (1,)), ((), ()))
    bc = _BI // _NSPLIT
    part = None
    for k in range(_NSPLIT):
        sl = slice(k * bc, (k + 1) * bc)
        wg = wg_ref[sl, :].astype(jnp.bfloat16)
        wu = wu_ref[sl, :].astype(jnp.bfloat16)
        wd = wd_ref[:, sl].astype(jnp.bfloat16)
        g = jax.lax.dot_general(xb, wg, nt, preferred_element_type=jnp.float32)
        u = jax.lax.dot_general(xb, wu, nt, preferred_element_type=jnp.float32)
        h = (g * jax.nn.sigmoid(g) * u).astype(jnp.bfloat16)
        p = jax.lax.dot_general(h, wd, nt, preferred_element_type=jnp.float32)
        part = p if part is None else part + p

    @pl.when(i == 0)
    def _init():
        out_ref[...] = part

    @pl.when(i > 0)
    def _acc():
        out_ref[...] += part


def _ffn_call(x2, Wg, Wu, Wd):
    M, H = x2.shape
    I = Wg.shape[0]
    return pl.pallas_call(
        _ffn_body,
        grid=(M // _BM, I // _BI),
        in_specs=[
            pl.BlockSpec((_BM, H), lambda m, i: (m, 0)),
            pl.BlockSpec((_BI, H), lambda m, i: (i, 0)),
            pl.BlockSpec((_BI, H), lambda m, i: (i, 0)),
            pl.BlockSpec((H, _BI), lambda m, i: (0, i)),
        ],
        out_specs=pl.BlockSpec((_BM, H), lambda m, i: (m, 0)),
        out_shape=jax.ShapeDtypeStruct((M, H), jnp.float32),
        scratch_shapes=[pltpu.VMEM((_BM, H), jnp.bfloat16)],
        compiler_params=pltpu.CompilerParams(
            dimension_semantics=("parallel", "arbitrary"),
        ),
    )(x2, Wg, Wu, Wd)


def kernel(x, Wg, Wu, Wd):
    B, S, H = x.shape
    M = B * S
    x2 = x.reshape(M, H)
    out = _ffn_call(x2, Wg, Wu, Wd)
    return out.reshape(B, S, H)


# speedup vs baseline: 1.0126x; 1.0126x over previous
import jax
import jax.numpy as jnp
from jax.experimental import pallas as pl
from jax.experimental.pallas import tpu as pltpu

_BM = 1024
_BI = 1024


_NSPLIT = 4


def _ffn_body(x_ref, wg_ref, wu_ref, wd_ref, out_ref):
    i = pl.program_id(1)
    xb = x_ref[...]
    nt = (((1,), (1,)), ((), ()))
    bc = _BI // _NSPLIT
    part = None
    for k in range(_NSPLIT):
        sl = slice(k * bc, (k + 1) * bc)
        wg = wg_ref[sl, :]
        wu = wu_ref[sl, :]
        wd = wd_ref[:, sl]
        g = jax.lax.dot_general(xb, wg, nt, preferred_element_type=jnp.float32)
        u = jax.lax.dot_general(xb, wu, nt, preferred_element_type=jnp.float32)
        h = g * jax.nn.sigmoid(g) * u
        p = jax.lax.dot_general(h, wd, nt, preferred_element_type=jnp.float32)
        part = p if part is None else part + p

    @pl.when(i == 0)
    def _init():
        out_ref[...] = part

    @pl.when(i > 0)
    def _acc():
        out_ref[...] += part


def _ffn_call(x2, Wg, Wu, Wd):
    M, H = x2.shape
    I = Wg.shape[0]
    return pl.pallas_call(
        _ffn_body,
        grid=(M // _BM, I // _BI),
        in_specs=[
            pl.BlockSpec((_BM, H), lambda m, i: (m, 0)),
            pl.BlockSpec((_BI, H), lambda m, i: (i, 0)),
            pl.BlockSpec((_BI, H), lambda m, i: (i, 0)),
            pl.BlockSpec((H, _BI), lambda m, i: (0, i)),
        ],
        out_specs=pl.BlockSpec((_BM, H), lambda m, i: (m, 0)),
        out_shape=jax.ShapeDtypeStruct((M, H), jnp.float32),
        compiler_params=pltpu.CompilerParams(
            dimension_semantics=("parallel", "arbitrary"),
        ),
    )(x2, Wg, Wu, Wd)


def kernel(x, Wg, Wu, Wd):
    B, S, H = x.shape
    M = B * S
    x2 = x.reshape(M, H)
    out = _ffn_call(x2, Wg, Wu, Wd)
    return out.reshape(B, S, H)


# speedup vs baseline: 1.0143x; 1.0017x over previous
import jax
import jax.numpy as jnp
from jax.experimental import pallas as pl
from jax.experimental.pallas import tpu as pltpu

_BM = 1024
_BI = 1024


_NSPLIT = 4


def _ffn_body(x_ref, wg_ref, wu_ref, wd_ref, out_ref):
    i = pl.program_id(1)
    xb = x_ref[...]
    nt = (((1,), (1,)), ((), ()))
    bc = _BI // _NSPLIT
    part = None
    for k in range(_NSPLIT):
        sl = slice(k * bc, (k + 1) * bc)
        wg = wg_ref[sl, :]
        wu = wu_ref[sl, :]
        wd = wd_ref[:, sl].astype(jnp.bfloat16)
        g = jax.lax.dot_general(xb, wg, nt, preferred_element_type=jnp.float32)
        u = jax.lax.dot_general(xb, wu, nt, preferred_element_type=jnp.float32)
        h = (g * jax.nn.sigmoid(g) * u).astype(jnp.bfloat16)
        p = jax.lax.dot_general(h, wd, nt, preferred_element_type=jnp.float32)
        part = p if part is None else part + p

    @pl.when(i == 0)
    def _init():
        out_ref[...] = part

    @pl.when(i > 0)
    def _acc():
        out_ref[...] += part


def _ffn_call(x2, Wg, Wu, Wd):
    M, H = x2.shape
    I = Wg.shape[0]
    return pl.pallas_call(
        _ffn_body,
        grid=(M // _BM, I // _BI),
        in_specs=[
            pl.BlockSpec((_BM, H), lambda m, i: (m, 0)),
            pl.BlockSpec((_BI, H), lambda m, i: (i, 0)),
            pl.BlockSpec((_BI, H), lambda m, i: (i, 0)),
            pl.BlockSpec((H, _BI), lambda m, i: (0, i)),
        ],
        out_specs=pl.BlockSpec((_BM, H), lambda m, i: (m, 0)),
        out_shape=jax.ShapeDtypeStruct((M, H), jnp.float32),
        compiler_params=pltpu.CompilerParams(
            dimension_semantics=("parallel", "arbitrary"),
        ),
    )(x2, Wg, Wu, Wd)


def kernel(x, Wg, Wu, Wd):
    B, S, H = x.shape
    M = B * S
    x2 = x.reshape(M, H)
    out = _ffn_call(x2, Wg, Wu, Wd)
    return out.reshape(B, S, H)
